# baseline (speedup 1.0000x reference)
import functools

import jax
import jax.numpy as jnp
from jax import lax
from jax.experimental import pallas as pl
from jax.experimental.pallas import tpu as pltpu
from jax.experimental.pallas import tpu_sc as plsc

_VOCAB = 100000
_DIM = 64
_CTX = 50
_HID = 128
_BLK = 16384

_NC = 2
_NW = 32
_SC_ROWS = 24576
_TC_ROWS = _VOCAB - _SC_ROWS
_RPW = _SC_ROWS // _NW
_CH = 256
_NCH = _RPW // _CH
_GRP = _CH // 16


def _h_body(idx_ref, emb_ref, w1_ref, b1_ref, h_ref, e_ref, sem):
    for c in range(_CTX):
        pltpu.make_async_copy(
            emb_ref.at[pl.ds(idx_ref[c], 1), :],
            e_ref.at[pl.ds(c, 1), :],
            sem,
        ).start()
    for c in range(_CTX):
        pltpu.make_async_copy(
            emb_ref.at[pl.ds(idx_ref[c], 1), :],
            e_ref.at[pl.ds(c, 1), :],
            sem,
        ).wait()
    h = b1_ref[...]
    for c in range(_CTX):
        h = h + lax.dot_general(
            e_ref[pl.ds(c, 1), :], w1_ref[:, c * _DIM:(c + 1) * _DIM],
            dimension_numbers=(((1,), (1,)), ((), ())),
            preferred_element_type=jnp.float32,
        )
    h_ref[...] = jnp.maximum(h, 0.0)


def _stage1(inputs, emb_table, W1, b1):
    return pl.pallas_call(
        _h_body,
        grid_spec=pltpu.PrefetchScalarGridSpec(
            num_scalar_prefetch=1,
            grid=(1,),
            in_specs=[
                pl.BlockSpec(memory_space=pltpu.MemorySpace.HBM),
                pl.BlockSpec((_HID, _CTX * _DIM), lambda i, idx: (0, 0)),
                pl.BlockSpec((1, _HID), lambda i, idx: (0, 0)),
            ],
            out_specs=pl.BlockSpec((1, _HID), lambda i, idx: (0, 0)),
            scratch_shapes=[
                pltpu.VMEM((_CTX, _DIM), jnp.float32),
                pltpu.SemaphoreType.DMA,
            ],
        ),
        out_shape=jax.ShapeDtypeStruct((1, _HID), jnp.float32),
    )(inputs, emb_table, W1, b1.reshape(1, _HID))


def _head_body(h_ref, w2_ref, b2_ref, out_ref):
    out_ref[...] = lax.dot_general(
        h_ref[...], w2_ref[...],
        dimension_numbers=(((1,), (1,)), ((), ())),
        preferred_element_type=jnp.float32,
    ) + b2_ref[...]


def _stage2_tc(h, W2, b2):
    return pl.pallas_call(
        _head_body,
        grid=(pl.cdiv(_TC_ROWS, _BLK),),
        in_specs=[
            pl.BlockSpec((1, _HID), lambda i: (0, 0)),
            pl.BlockSpec((_BLK, _HID), lambda i: (i, 0)),
            pl.BlockSpec((1, _BLK), lambda i: (0, i)),
        ],
        out_specs=pl.BlockSpec((1, _BLK), lambda i: (0, i)),
        out_shape=jax.ShapeDtypeStruct((1, _TC_ROWS), jnp.float32),
    )(h, W2, b2.reshape(1, _VOCAB))


def _sc_tail_body(w2_hbm, b2_hbm, h_hbm, out_hbm,
                  h_v, b2_v, w2_a, w2_b, out_v, sem_a, sem_b):
    wid = lax.axis_index("s") * _NC + lax.axis_index("c")
    base = _TC_ROWS + wid * _RPW
    pltpu.sync_copy(h_hbm, h_v)
    pltpu.sync_copy(b2_hbm.at[pl.ds(base, _RPW)], b2_v)

    lane = lax.iota(jnp.int32, 16)

    def w2_copy(ic, buf, sem):
        return pltpu.make_async_copy(
            w2_hbm.at[pl.ds(base + ic * _CH, _CH), :], buf, sem)

    w2_copy(0, w2_a, sem_a).start()
    for ic in range(_NCH):
        cur = w2_a if ic % 2 == 0 else w2_b
        csem = sem_a if ic % 2 == 0 else sem_b
        if ic + 1 < _NCH:
            nxt = w2_b if ic % 2 == 0 else w2_a
            nsem = sem_b if ic % 2 == 0 else sem_a
            w2_copy(ic + 1, nxt, nsem).start()
        w2_copy(ic, cur, csem).wait()

        h_chunks = [h_v[pl.ds(16 * j, 16)] for j in range(_HID // 16)]

        def group(g, carry):
            r0 = g * 16
            sums = b2_v[pl.ds(ic * _CH + r0, 16)]
            for i in range(16):
                ps = [cur[r0 + i, pl.ds(16 * j, 16)] * h_chunks[j]
                      for j in range(_HID // 16)]
                acc = ((ps[0] + ps[1]) + (ps[2] + ps[3])) + (
                    (ps[4] + ps[5]) + (ps[6] + ps[7]))
                sums = jnp.where(lane == i, sums + jnp.sum(acc), sums)
            out_v[pl.ds(ic * _CH + r0, 16)] = sums
            return carry

        lax.fori_loop(0, _GRP, group, 0)
    pltpu.sync_copy(out_v, out_hbm.at[pl.ds(wid * _RPW, _RPW)])


@functools.cache
def _make_sc_tail():
    mesh = plsc.VectorSubcoreMesh(core_axis_name="c", subcore_axis_name="s")
    return functools.partial(
        pl.kernel,
        mesh=mesh,
        compiler_params=pltpu.CompilerParams(needs_layout_passes=False),
        out_type=jax.ShapeDtypeStruct((_SC_ROWS,), jnp.float32),
        scratch_types=[
            pltpu.VMEM((_HID,), jnp.float32),
            pltpu.VMEM((_RPW,), jnp.float32),
            pltpu.VMEM((_CH, _HID), jnp.float32),
            pltpu.VMEM((_CH, _HID), jnp.float32),
            pltpu.VMEM((_RPW,), jnp.float32),
            pltpu.SemaphoreType.DMA,
            pltpu.SemaphoreType.DMA,
        ],
    )(_sc_tail_body)


def kernel(inputs, emb_table, W1, b1, W2, b2):
    h = _stage1(inputs, emb_table, W1, b1)
    out_sc = _make_sc_tail()(W2, b2, h.reshape(_HID))
    out_tc = _stage2_tc(h, W2, b2)
    return jnp.concatenate([out_tc, out_sc.reshape(1, _SC_ROWS)], axis=1)


# speedup vs baseline: 1.2867x; 1.2867x over previous
import functools

import jax
import jax.numpy as jnp
from jax import lax
from jax.experimental import pallas as pl
from jax.experimental.pallas import tpu as pltpu

_VOCAB = 100000
_DIM = 64
_CTX = 50
_HID = 128
_BLK = 16384


def _mlp_body(idx_ref, emb_ref, w1_ref, b1_ref, w2_ref, b2_ref, out_ref,
              e_ref, h_ref, sem):
    i = pl.program_id(0)

    @pl.when(i == 0)
    def _():
        for c in range(_CTX):
            pltpu.make_async_copy(
                emb_ref.at[pl.ds(idx_ref[c], 1), :],
                e_ref.at[pl.ds(c, 1), :],
                sem,
            ).start()
        for c in range(_CTX):
            pltpu.make_async_copy(
                emb_ref.at[pl.ds(idx_ref[c], 1), :],
                e_ref.at[pl.ds(c, 1), :],
                sem,
            ).wait()
        h = b1_ref[...]
        for c in range(_CTX):
            h = h + lax.dot_general(
                e_ref[pl.ds(c, 1), :], w1_ref[:, c * _DIM:(c + 1) * _DIM],
                dimension_numbers=(((1,), (1,)), ((), ())),
                preferred_element_type=jnp.float32,
            )
        h_ref[...] = jnp.maximum(h, 0.0)

    out_ref[...] = lax.dot_general(
        h_ref[...], w2_ref[...],
        dimension_numbers=(((1,), (1,)), ((), ())),
        preferred_element_type=jnp.float32,
    ) + b2_ref[...]


def kernel(inputs, emb_table, W1, b1, W2, b2):
    grid = (pl.cdiv(_VOCAB, _BLK),)
    return pl.pallas_call(
        _mlp_body,
        grid_spec=pltpu.PrefetchScalarGridSpec(
            num_scalar_prefetch=1,
            grid=grid,
            in_specs=[
                pl.BlockSpec(memory_space=pltpu.MemorySpace.HBM),
                pl.BlockSpec((_HID, _CTX * _DIM), lambda i, idx: (0, 0)),
                pl.BlockSpec((1, _HID), lambda i, idx: (0, 0)),
                pl.BlockSpec((_BLK, _HID), lambda i, idx: (i, 0)),
                pl.BlockSpec((1, _BLK), lambda i, idx: (0, i)),
            ],
            out_specs=pl.BlockSpec((1, _BLK), lambda i, idx: (0, i)),
            scratch_shapes=[
                pltpu.VMEM((_CTX, _DIM), jnp.float32),
                pltpu.VMEM((1, _HID), jnp.float32),
                pltpu.SemaphoreType.DMA,
            ],
        ),
        out_shape=jax.ShapeDtypeStruct((1, _VOCAB), jnp.float32),
    )(inputs, emb_table, W1, b1.reshape(1, _HID), W2, b2.reshape(1, _VOCAB))
